# baseline (speedup 1.0000x reference)
import jax
import jax.numpy as jnp
from jax import lax
from jax.experimental import pallas as pl
from jax.experimental.pallas import tpu as pltpu
from jax.experimental.pallas import tpu_sc as plsc

_B = 4096
_F = 26
_V = 100000
_D = 16
_L = 50
_ND = 13
_NW = 32
_BPW = _B // _NW
_EC = (_BPW * _F) // 128
_SCH = 64
_SRING = 4


_VP = 100096
_VBLK = 4352
_NVB = _VP // _VBLK


def _dt_body(x_ref, o_ref):
    v = pl.program_id(1)
    x = x_ref[:, pl.ds(v * _VBLK, _VBLK)]
    o_ref[...] = x.T


def _detile(x, groups):
    return pl.pallas_call(
        _dt_body,
        grid=(groups, _NVB),
        in_specs=[pl.BlockSpec((_D, _V), lambda f, v: (f, 0))],
        out_specs=pl.BlockSpec((_VBLK, _D), lambda f, v: (f * _NVB + v, 0)),
        out_shape=jax.ShapeDtypeStruct((groups * _VP, _D), jnp.float32),
    )(x)


def _sc_body(emb_hbm, lin_hbm, seqtab_hbm, eidx_hbm, sidx_hbm,
             out_g, out_seq, out_lin,
             eidx_v, sidx_v, ebuf, lbuf, sbuf, sacc_v,
             sem_e, sem_l, sem_s0, sem_s1, sem_s2, sem_s3):
    w = lax.axis_index("s") * 2 + lax.axis_index("c")

    pltpu.sync_copy(eidx_hbm.at[w], eidx_v)
    pltpu.sync_copy(sidx_hbm.at[w], sidx_v)

    for c in range(_EC):
        pltpu.async_copy(emb_hbm.at[eidx_v.at[c]],
                         ebuf.at[pl.ds(c * 128, 128)], sem_e)
    for c in range(_F):
        pltpu.async_copy(lin_hbm.at[eidx_v.at[c]],
                         lbuf.at[pl.ds(c * 128, 128)], sem_l)

    sems = (sem_s0, sem_s1, sem_s2, sem_s3)
    for r in range(_SRING):
        pltpu.async_copy(seqtab_hbm.at[sidx_v.at[r]], sbuf.at[r], sems[r])

    inv_l = jnp.full((_D,), 1.0 / _L, jnp.float32)

    def _seq_chunk(c, r):
        pltpu.make_async_copy(seqtab_hbm.at[sidx_v.at[0]], sbuf.at[r],
                              sems[r]).wait()
        a0 = sbuf[r, 0]
        a1 = sbuf[r, _L]
        for t in range(1, _L):
            a0 = a0 + sbuf[r, t]
            a1 = a1 + sbuf[r, _L + t]
        sacc_v[2 * c] = a0 * inv_l
        sacc_v[2 * c + 1] = a1 * inv_l

    def _ring_body(i, carry):
        for r in range(_SRING):
            c = i * _SRING + r
            _seq_chunk(c, r)
            pltpu.async_copy(seqtab_hbm.at[sidx_v.at[c + _SRING]],
                             sbuf.at[r], sems[r])
        return carry

    lax.fori_loop(0, _SCH // _SRING - 1, _ring_body, 0)
    for r in range(_SRING):
        _seq_chunk(_SCH - _SRING + r, r)
    pltpu.sync_copy(sacc_v, out_seq.at[pl.ds(w * _BPW, _BPW)])

    for c in range(_F):
        pltpu.make_async_copy(lin_hbm.at[eidx_v.at[0]],
                              lbuf.at[pl.ds(c * 128, 128)], sem_l).wait()
    pltpu.sync_copy(lbuf, out_lin.at[pl.ds(w * (_F * 128), _F * 128)])

    for c in range(_EC):
        pltpu.make_async_copy(emb_hbm.at[eidx_v.at[0]],
                              ebuf.at[pl.ds(c * 128, 128)], sem_e).wait()
    pltpu.sync_copy(ebuf, out_g.at[pl.ds(w * _BPW * _F, _BPW * _F)])


_sc_gather = pl.kernel(
    _sc_body,
    mesh=plsc.VectorSubcoreMesh(core_axis_name="c", subcore_axis_name="s"),
    compiler_params=pltpu.CompilerParams(use_tc_tiling_on_sc=False),
    out_type=[
        jax.ShapeDtypeStruct((_B * _F, _D), jnp.float32),
        jax.ShapeDtypeStruct((_B, _D), jnp.float32),
        jax.ShapeDtypeStruct((_B * _F,), jnp.float32),
    ],
    scratch_types=[
        pltpu.VMEM((_EC, 128), jnp.int32),
        pltpu.VMEM((_SCH, 2 * _L), jnp.int32),
        pltpu.VMEM((_BPW * _F, _D), jnp.float32),
        pltpu.VMEM((_F * 128,), jnp.float32),
        pltpu.VMEM((_SRING, 2 * _L, _D), jnp.float32),
        pltpu.VMEM((_BPW, _D), jnp.float32),
        pltpu.SemaphoreType.DMA,
        pltpu.SemaphoreType.DMA,
        pltpu.SemaphoreType.DMA,
        pltpu.SemaphoreType.DMA,
        pltpu.SemaphoreType.DMA,
        pltpu.SemaphoreType.DMA,
    ],
)


_BLK = 512


def _tc_body(consts_ref, dense_ref, g_ref, seq_ref, lin_ref, wld_ref,
             w1a_ref, w1b_ref, w1c_ref, b1_ref, w2_ref, b2_ref, w3_ref,
             o_ref):
    f32 = jnp.float32
    g = g_ref[...]
    seq = seq_ref[...]
    dense = dense_ref[...]
    jrow = lax.broadcasted_iota(jnp.int32, (_F * _D, _D), 0)
    dcol = lax.broadcasted_iota(jnp.int32, (_F * _D, _D), 1)
    smat = jnp.where(jrow % _D == dcol, 1.0, 0.0).astype(f32)
    s = jnp.dot(g, smat, preferred_element_type=f32) + seq
    sumsq = (jnp.sum(g * g, axis=1, keepdims=True)
             + jnp.sum(seq * seq, axis=1, keepdims=True))
    fm = 0.5 * (jnp.sum(s * s, axis=1, keepdims=True) - sumsq)
    lin_sum = jnp.sum(lin_ref[...], axis=1, keepdims=True)
    linear = (jnp.dot(dense, wld_ref[...], preferred_element_type=f32)
              + consts_ref[0] + lin_sum)
    h = jnp.dot(dense, w1a_ref[...], preferred_element_type=f32)
    h = h + jnp.dot(g, w1b_ref[...], preferred_element_type=f32)
    h = h + jnp.dot(seq, w1c_ref[...], preferred_element_type=f32)
    h = jnp.maximum(h + b1_ref[...], 0.0)
    h = jnp.maximum(jnp.dot(h, w2_ref[...], preferred_element_type=f32)
                    + b2_ref[...], 0.0)
    dnn = jnp.dot(h, w3_ref[...], preferred_element_type=f32) + consts_ref[1]
    logit = linear + fm + dnn
    o_ref[...] = jax.nn.sigmoid(logit * consts_ref[3] + consts_ref[2])


def kernel(dense, sparse, sequence, emb_tables, lin_tables, seq_table,
           W_ld, b_ld, W1, b1, W2, b2, W3, b3, W_out, b_out):
    emb_slab = emb_tables.transpose(0, 2, 1).reshape(_F * _D, _V)
    emb_dm = _detile(emb_slab, _F)
    seq_dm = _detile(seq_table.T, 1)
    lin_vm = jnp.pad(jnp.sum(lin_tables, axis=2),
                     ((0, 0), (0, _VP - _V))).reshape(_F * _VP)

    offs = (jnp.arange(_F, dtype=jnp.int32) * _VP).reshape(1, _F)
    eidx = (sparse + offs).reshape(_NW, _EC, 128)
    sidx = sequence.reshape(_NW, _SCH, 2 * _L)

    out_g, out_seq, out_lin = _sc_gather(emb_dm, lin_vm, seq_dm, eidx, sidx)

    g2 = out_g.reshape(_B, _F * _D)
    lin2 = out_lin.reshape(_B, _F)
    consts = jnp.concatenate([b_ld, b3, b_out, W_out.reshape(1)])
    w1a = W1[0:_ND]
    w1b = W1[_ND:_ND + _F * _D]
    w1c = W1[_ND + _F * _D:]
    b1r = b1.reshape(1, 128)
    b2r = b2.reshape(1, 128)

    return pl.pallas_call(
        _tc_body,
        grid=(_B // _BLK,),
        in_specs=[
            pl.BlockSpec(memory_space=pltpu.SMEM),
            pl.BlockSpec((_BLK, _ND), lambda i: (i, 0)),
            pl.BlockSpec((_BLK, _F * _D), lambda i: (i, 0)),
            pl.BlockSpec((_BLK, _D), lambda i: (i, 0)),
            pl.BlockSpec((_BLK, _F), lambda i: (i, 0)),
            pl.BlockSpec((_ND, 1), lambda i: (0, 0)),
            pl.BlockSpec((_ND, 128), lambda i: (0, 0)),
            pl.BlockSpec((_F * _D, 128), lambda i: (0, 0)),
            pl.BlockSpec((_D, 128), lambda i: (0, 0)),
            pl.BlockSpec((1, 128), lambda i: (0, 0)),
            pl.BlockSpec((128, 128), lambda i: (0, 0)),
            pl.BlockSpec((1, 128), lambda i: (0, 0)),
            pl.BlockSpec((128, 1), lambda i: (0, 0)),
        ],
        out_specs=pl.BlockSpec((_BLK, 1), lambda i: (i, 0)),
        out_shape=jax.ShapeDtypeStruct((_B, 1), jnp.float32),
    )(consts, dense, g2, out_seq, lin2, W_ld, w1a, w1b, w1c, b1r, W2, b2r, W3)


# speedup vs baseline: 1.9074x; 1.9074x over previous
import jax
import jax.numpy as jnp
from jax import lax
from jax.experimental import pallas as pl
from jax.experimental.pallas import tpu as pltpu
from jax.experimental.pallas import tpu_sc as plsc

_B = 4096
_F = 26
_V = 100000
_D = 16
_L = 50
_LP = 56
_ND = 13
_NW = 32
_BPW = _B // _NW

_EC = (_BPW * _F * _D) // 128
_EWAVES = 8
_EWC = _EC // _EWAVES
_SC = (_BPW * _LP * _D) // 128
_SWAVES = 8
_SWC = _SC // _SWAVES
_SBW = _BPW // _SWAVES


def _sc_body(emb_hbm, lin_hbm, seq_hbm, eidx_hbm, lidx_hbm, sidx_hbm,
             out_g, out_seq, out_lin,
             eidx_v, ebuf, sidx_v, sbuf, lidx_v, lbuf, sacc_v,
             sem_l, sem_ei0, sem_ei1, sem_eg0, sem_eg1,
             sem_si0, sem_si1, sem_sg0, sem_sg1):
    w = lax.axis_index("s") * 2 + lax.axis_index("c")
    sem_ei = (sem_ei0, sem_ei1)
    sem_eg = (sem_eg0, sem_eg1)
    sem_si = (sem_si0, sem_si1)
    sem_sg = (sem_sg0, sem_sg1)

    pltpu.sync_copy(lidx_hbm.at[w], lidx_v)
    for c in range(_F):
        pltpu.async_copy(lin_hbm.at[lidx_v.at[c]],
                         lbuf.at[pl.ds(c * 128, 128)], sem_l)

    pltpu.async_copy(eidx_hbm.at[w, pl.ds(0, _EWC)], eidx_v.at[0], sem_ei0)
    pltpu.async_copy(sidx_hbm.at[w, pl.ds(0, _SWC)], sidx_v.at[0], sem_si0)

    def _emb_half(k, s):
        pltpu.make_async_copy(eidx_hbm.at[w, pl.ds(0, _EWC)],
                              eidx_v.at[s], sem_ei[s]).wait()
        for c in range(_EWC):
            pltpu.async_copy(emb_hbm.at[eidx_v.at[s, c]],
                             ebuf.at[s, pl.ds(c * 128, 128)],
                             sem_eg[s])

        @pl.when(k > 0)
        def _():
            for c in range(_EWC):
                pltpu.make_async_copy(
                    emb_hbm.at[eidx_v.at[1 - s, 0]],
                    ebuf.at[1 - s, pl.ds(c * 128, 128)], sem_eg[1 - s]).wait()
            pltpu.sync_copy(
                ebuf.at[1 - s],
                out_g.at[pl.ds(w * (_EC * 128) + (k - 1) * (_EWC * 128),
                               _EWC * 128)])

        @pl.when(k + 1 < _EWAVES)
        def _():
            pltpu.async_copy(eidx_hbm.at[w, pl.ds((k + 1) * _EWC, _EWC)],
                             eidx_v.at[1 - s], sem_ei[1 - s])

    def _emb_pair(i, carry):
        _emb_half(2 * i, 0)
        _emb_half(2 * i + 1, 1)
        return carry

    lax.fori_loop(0, _EWAVES // 2, _emb_pair, 0)
    for c in range(_EWC):
        pltpu.make_async_copy(emb_hbm.at[eidx_v.at[1, 0]],
                              ebuf.at[1, pl.ds(c * 128, 128)],
                              sem_eg[1]).wait()
    pltpu.sync_copy(
        ebuf.at[1],
        out_g.at[pl.ds(w * (_EC * 128) + (_EWAVES - 1) * (_EWC * 128),
                       _EWC * 128)])

    inv_l = jnp.full((_D,), 1.0 / _L, jnp.float32)

    def _seq_wave(k, carry):
        slot = lax.rem(k, 2)
        pltpu.make_async_copy(sidx_hbm.at[w, pl.ds(0, _SWC)],
                              sidx_v.at[slot], sem_si[0]).wait()

        @pl.when(k < _SWAVES - 1)
        def _():
            pltpu.async_copy(sidx_hbm.at[w, pl.ds((k + 1) * _SWC, _SWC)],
                             sidx_v.at[1 - slot], sem_si[0])

        for c in range(_SWC):
            pltpu.async_copy(seq_hbm.at[sidx_v.at[slot, c]],
                             sbuf.at[slot, pl.ds(c * 128, 128)],
                             sem_sg[0])
        for c in range(_SWC):
            pltpu.make_async_copy(seq_hbm.at[sidx_v.at[slot, 0]],
                                  sbuf.at[slot, pl.ds(c * 128, 128)],
                                  sem_sg[0]).wait()
        for b in range(_SBW):
            base = b * (_LP * _D)
            a = sbuf[slot, pl.ds(base, _D)]
            for t in range(1, _L):
                a = a + sbuf[slot, pl.ds(base + t * _D, _D)]
            sacc_v[k * _SBW + b] = a * inv_l
        return carry

    lax.fori_loop(0, _SWAVES, _seq_wave, 0)
    pltpu.sync_copy(sacc_v, out_seq.at[pl.ds(w * _BPW, _BPW)])

    for c in range(_F):
        pltpu.make_async_copy(lin_hbm.at[lidx_v.at[0]],
                              lbuf.at[pl.ds(c * 128, 128)], sem_l).wait()
    pltpu.sync_copy(lbuf, out_lin.at[pl.ds(w * (_F * 128), _F * 128)])


_sc_gather = pl.kernel(
    _sc_body,
    mesh=plsc.VectorSubcoreMesh(core_axis_name="c", subcore_axis_name="s"),
    compiler_params=pltpu.CompilerParams(use_tc_tiling_on_sc=False),
    out_type=[
        jax.ShapeDtypeStruct((_B * _F * _D,), jnp.float32),
        jax.ShapeDtypeStruct((_B, _D), jnp.float32),
        jax.ShapeDtypeStruct((_B * _F,), jnp.float32),
    ],
    scratch_types=[
        pltpu.VMEM((2, _EWC, 128), jnp.int32),
        pltpu.VMEM((2, _EWC * 128), jnp.float32),
        pltpu.VMEM((2, _SWC, 128), jnp.int32),
        pltpu.VMEM((2, _SWC * 128), jnp.float32),
        pltpu.VMEM((_F, 128), jnp.int32),
        pltpu.VMEM((_F * 128,), jnp.float32),
        pltpu.VMEM((_BPW, _D), jnp.float32),
        pltpu.SemaphoreType.DMA,
        pltpu.SemaphoreType.DMA,
        pltpu.SemaphoreType.DMA,
        pltpu.SemaphoreType.DMA,
        pltpu.SemaphoreType.DMA,
        pltpu.SemaphoreType.DMA,
        pltpu.SemaphoreType.DMA,
        pltpu.SemaphoreType.DMA,
        pltpu.SemaphoreType.DMA,
    ],
)


_BLK = 512


def _tc_body(consts_ref, dense_ref, g_ref, seq_ref, lin_ref, wld_ref,
             w1a_ref, w1b_ref, w1c_ref, b1_ref, w2_ref, b2_ref, w3_ref,
             o_ref):
    f32 = jnp.float32
    g = g_ref[...]
    seq = seq_ref[...]
    dense = dense_ref[...]
    jrow = lax.broadcasted_iota(jnp.int32, (_F * _D, _D), 0)
    dcol = lax.broadcasted_iota(jnp.int32, (_F * _D, _D), 1)
    smat = jnp.where(jrow % _D == dcol, 1.0, 0.0).astype(f32)
    s = jnp.dot(g, smat, preferred_element_type=f32) + seq
    sumsq = (jnp.sum(g * g, axis=1, keepdims=True)
             + jnp.sum(seq * seq, axis=1, keepdims=True))
    fm = 0.5 * (jnp.sum(s * s, axis=1, keepdims=True) - sumsq)
    lin_sum = jnp.sum(lin_ref[...], axis=1, keepdims=True)
    linear = (jnp.dot(dense, wld_ref[...], preferred_element_type=f32)
              + consts_ref[0] + lin_sum)
    h = jnp.dot(dense, w1a_ref[...], preferred_element_type=f32)
    h = h + jnp.dot(g, w1b_ref[...], preferred_element_type=f32)
    h = h + jnp.dot(seq, w1c_ref[...], preferred_element_type=f32)
    h = jnp.maximum(h + b1_ref[...], 0.0)
    h = jnp.maximum(jnp.dot(h, w2_ref[...], preferred_element_type=f32)
                    + b2_ref[...], 0.0)
    dnn = jnp.dot(h, w3_ref[...], preferred_element_type=f32) + consts_ref[1]
    logit = linear + fm + dnn
    o_ref[...] = jax.nn.sigmoid(logit * consts_ref[3] + consts_ref[2])


def kernel(dense, sparse, sequence, emb_tables, lin_tables, seq_table,
           W_ld, b_ld, W1, b1, W2, b2, W3, b3, W_out, b_out):
    emb_vm = emb_tables.transpose(0, 2, 1).reshape(_F * _D * _V)
    seq_vm = seq_table.T.reshape(_D * _V)
    lin_vm = jnp.sum(lin_tables, axis=2).reshape(_F * _V)

    plane_e = (jnp.arange(_F * _D, dtype=jnp.int32) * _V).reshape(1, _F, _D)
    eidx = (sparse[:, :, None] + plane_e).reshape(_NW, _EC, 128)
    plane_s = (jnp.arange(_D, dtype=jnp.int32) * _V).reshape(1, 1, _D)
    seqp = jnp.pad(sequence, ((0, 0), (0, _LP - _L)))
    sidx = (seqp[:, :, None] + plane_s).reshape(_NW, _SC, 128)
    offs = (jnp.arange(_F, dtype=jnp.int32) * _V).reshape(1, _F)
    lidx = (sparse + offs).reshape(_NW, _F, 128)

    out_g, out_seq, out_lin = _sc_gather(emb_vm, lin_vm, seq_vm,
                                         eidx, lidx, sidx)

    g2 = out_g.reshape(_B, _F * _D)
    lin2 = out_lin.reshape(_B, _F)
    consts = jnp.concatenate([b_ld, b3, b_out, W_out.reshape(1)])
    w1a = W1[0:_ND]
    w1b = W1[_ND:_ND + _F * _D]
    w1c = W1[_ND + _F * _D:]
    b1r = b1.reshape(1, 128)
    b2r = b2.reshape(1, 128)

    return pl.pallas_call(
        _tc_body,
        grid=(_B // _BLK,),
        in_specs=[
            pl.BlockSpec(memory_space=pltpu.SMEM),
            pl.BlockSpec((_BLK, _ND), lambda i: (i, 0)),
            pl.BlockSpec((_BLK, _F * _D), lambda i: (i, 0)),
            pl.BlockSpec((_BLK, _D), lambda i: (i, 0)),
            pl.BlockSpec((_BLK, _F), lambda i: (i, 0)),
            pl.BlockSpec((_ND, 1), lambda i: (0, 0)),
            pl.BlockSpec((_ND, 128), lambda i: (0, 0)),
            pl.BlockSpec((_F * _D, 128), lambda i: (0, 0)),
            pl.BlockSpec((_D, 128), lambda i: (0, 0)),
            pl.BlockSpec((1, 128), lambda i: (0, 0)),
            pl.BlockSpec((128, 128), lambda i: (0, 0)),
            pl.BlockSpec((1, 128), lambda i: (0, 0)),
            pl.BlockSpec((128, 1), lambda i: (0, 0)),
        ],
        out_specs=pl.BlockSpec((_BLK, 1), lambda i: (i, 0)),
        out_shape=jax.ShapeDtypeStruct((_B, 1), jnp.float32),
    )(consts, dense, g2, out_seq, lin2, W_ld, w1a, w1b, w1c, b1r, W2, b2r, W3)
